# baseline (speedup 1.0000x reference)
import functools

import jax
import jax.numpy as jnp
from jax.experimental import pallas as pl
from jax.experimental.pallas import tpu as pltpu

_D = 768
_MAX_REL = 125
_VOCAB = 2 * _MAX_REL + 1
_CHS = 256
_NB = 4


def _body(x_hbm, table_ref, out_hbm, xbuf, obuf, insems, outsems, *, S, B):
    nch = S // _CHS
    tbl = table_ref[...]

    def load(i):
        sl = i % _NB
        return [pltpu.make_async_copy(
            x_hbm.at[b, pl.ds(i * _CHS, _CHS), :],
            xbuf.at[sl, b], insems.at[sl, b])
            for b in range(B)]

    def store(i):
        sl = i % _NB
        return [pltpu.make_async_copy(
            obuf.at[sl, b], out_hbm.at[b, pl.ds(i * _CHS, _CHS), :],
            outsems.at[sl, b])
            for b in range(B)]

    for i in range(min(_NB, nch)):
        for cp in load(i):
            cp.start()

    for i in range(nch):
        sl = i % _NB
        jj = i * _CHS + jax.lax.broadcasted_iota(jnp.int32, (_CHS, _VOCAB), 0)
        kk = jax.lax.broadcasted_iota(jnp.int32, (_CHS, _VOCAB), 1)
        interior = ((kk >= 1) & (kk <= _VOCAB - 2)
                    & (jj >= kk - _MAX_REL) & (jj <= kk - _MAX_REL + S - 1))
        cnt = jnp.where(kk == 0, jnp.maximum(0, (S - _MAX_REL) - jj), 0)
        cnt = cnt + jnp.where(kk == _VOCAB - 1,
                              jnp.maximum(0, jj - (_MAX_REL - 1)), 0)
        cnt = cnt + interior.astype(jnp.int32)
        c = cnt.astype(jnp.float32) * (1.0 / S)
        pooled = jax.lax.dot_general(
            c, tbl,
            dimension_numbers=(((1,), (0,)), ((), ())),
            preferred_element_type=jnp.float32,
        )
        for cp in load(i):
            cp.wait()
        if i >= _NB:
            for cp in store(i - _NB):
                cp.wait()
        obuf[sl] = xbuf[sl] + pooled[None, :, :]
        for cp in store(i):
            cp.start()
        if i + _NB < nch:
            for cp in load(i + _NB):
                cp.start()

    for i in range(max(0, nch - _NB), nch):
        for cp in store(i):
            cp.wait()


def kernel(x, table):
    B, S, d = x.shape
    V = table.shape[0]
    body = functools.partial(_body, S=S, B=B)
    return pl.pallas_call(
        body,
        in_specs=[
            pl.BlockSpec(memory_space=pl.ANY),
            pl.BlockSpec((V, d), lambda: (0, 0)),
        ],
        out_specs=pl.BlockSpec(memory_space=pl.ANY),
        out_shape=jax.ShapeDtypeStruct((B, S, d), x.dtype),
        scratch_shapes=[
            pltpu.VMEM((_NB, 2, _CHS, d), jnp.float32),
            pltpu.VMEM((_NB, 2, _CHS, d), jnp.float32),
            pltpu.SemaphoreType.DMA((_NB, 2)),
            pltpu.SemaphoreType.DMA((_NB, 2)),
        ],
    )(x, table)


# speedup vs baseline: 1.1070x; 1.1070x over previous
import functools

import jax
import jax.numpy as jnp
from jax.experimental import pallas as pl
from jax.experimental.pallas import tpu as pltpu

_D = 768
_MAX_REL = 125
_VOCAB = 2 * _MAX_REL + 1
_CHS = 512
_NB = 4


def _body(x_hbm, table_ref, out_hbm, xbuf, obuf, insems, outsems, *, S, B):
    nch = S // _CHS
    tbl = table_ref[...]

    def load(i):
        sl = i % _NB
        return [pltpu.make_async_copy(
            x_hbm.at[b, pl.ds(i * _CHS, _CHS), :],
            xbuf.at[sl, b], insems.at[sl, b])
            for b in range(B)]

    def store(i):
        sl = i % _NB
        return [pltpu.make_async_copy(
            obuf.at[sl, b], out_hbm.at[b, pl.ds(i * _CHS, _CHS), :],
            outsems.at[sl, b])
            for b in range(B)]

    for i in range(min(_NB, nch)):
        for cp in load(i):
            cp.start()

    for i in range(nch):
        sl = i % _NB
        jj = i * _CHS + jax.lax.broadcasted_iota(jnp.int32, (_CHS, _VOCAB), 0)
        kk = jax.lax.broadcasted_iota(jnp.int32, (_CHS, _VOCAB), 1)
        interior = ((kk >= 1) & (kk <= _VOCAB - 2)
                    & (jj >= kk - _MAX_REL) & (jj <= kk - _MAX_REL + S - 1))
        cnt = jnp.where(kk == 0, jnp.maximum(0, (S - _MAX_REL) - jj), 0)
        cnt = cnt + jnp.where(kk == _VOCAB - 1,
                              jnp.maximum(0, jj - (_MAX_REL - 1)), 0)
        cnt = cnt + interior.astype(jnp.int32)
        c = cnt.astype(jnp.float32) * (1.0 / S)
        pooled = jax.lax.dot_general(
            c, tbl,
            dimension_numbers=(((1,), (0,)), ((), ())),
            preferred_element_type=jnp.float32,
        )
        for cp in load(i):
            cp.wait()
        if i >= _NB:
            for cp in store(i - _NB):
                cp.wait()
        obuf[sl] = xbuf[sl] + pooled[None, :, :]
        for cp in store(i):
            cp.start()
        if i + _NB < nch:
            for cp in load(i + _NB):
                cp.start()

    for i in range(max(0, nch - _NB), nch):
        for cp in store(i):
            cp.wait()


def kernel(x, table):
    B, S, d = x.shape
    V = table.shape[0]
    body = functools.partial(_body, S=S, B=B)
    return pl.pallas_call(
        body,
        in_specs=[
            pl.BlockSpec(memory_space=pl.ANY),
            pl.BlockSpec((V, d), lambda: (0, 0)),
        ],
        out_specs=pl.BlockSpec(memory_space=pl.ANY),
        out_shape=jax.ShapeDtypeStruct((B, S, d), x.dtype),
        scratch_shapes=[
            pltpu.VMEM((_NB, 2, _CHS, d), jnp.float32),
            pltpu.VMEM((_NB, 2, _CHS, d), jnp.float32),
            pltpu.SemaphoreType.DMA((_NB, 2)),
            pltpu.SemaphoreType.DMA((_NB, 2)),
        ],
    )(x, table)


# speedup vs baseline: 1.1958x; 1.0802x over previous
import functools

import jax
import jax.numpy as jnp
from jax.experimental import pallas as pl

_D = 768
_MAX_REL = 125
_VOCAB = 2 * _MAX_REL + 1
_BLK = 1024


def _body(x_ref, table_ref, out_ref, *, S):
    s = pl.program_id(0)
    blk = out_ref.shape[1]
    kdim = table_ref.shape[0]
    jj = s * blk + jax.lax.broadcasted_iota(jnp.int32, (blk, kdim), 0)
    kk = jax.lax.broadcasted_iota(jnp.int32, (blk, kdim), 1)
    km = kk - _MAX_REL
    hi = jnp.where(kk == _VOCAB - 1, S + _MAX_REL, km)
    lo = jnp.where(kk == 0, -(S + _MAX_REL), km)
    cnt = jnp.maximum(0, jnp.minimum(jj, hi) - jnp.maximum(jj - (S - 1), lo) + 1)
    c = cnt.astype(jnp.float32) * (1.0 / S)
    pooled = jax.lax.dot_general(
        c, table_ref[...],
        dimension_numbers=(((1,), (0,)), ((), ())),
        preferred_element_type=jnp.float32,
    )
    out_ref[...] = x_ref[...] + pooled[None, :, :]


def kernel(x, table):
    B, S, d = x.shape
    V = table.shape[0]
    grid = (S // _BLK,)
    body = functools.partial(_body, S=S)
    return pl.pallas_call(
        body,
        grid=grid,
        in_specs=[
            pl.BlockSpec((B, _BLK, d), lambda s: (0, s, 0)),
            pl.BlockSpec((V, d), lambda s: (0, 0)),
        ],
        out_specs=pl.BlockSpec((B, _BLK, d), lambda s: (0, s, 0)),
        out_shape=jax.ShapeDtypeStruct((B, S, d), x.dtype),
    )(x, table)
